# Initial kernel scaffold; baseline (speedup 1.0000x reference)
import jax
import jax.numpy as jnp
from jax.experimental import pallas as pl


def kernel(images, predictions):
    raise NotImplementedError("write your pallas kernel here")


# speedup vs baseline: 1.3275x; 1.3275x over previous
import math

import jax
import jax.numpy as jnp
import numpy as np
from jax.experimental import pallas as pl
from jax.experimental.pallas import tpu as pltpu

_NUM_CLASSES = 18
_CONF_THRESH = 0.05
_IOU_THRESH = 0.5
_K = 500
_S = 512
_K3 = 1024
_MAX_TOTAL = 1000
_A = 49104
_AP = 49152
_R = 384
_M3 = _NUM_CLASSES * _S
_R3 = 72
_NEG = -1.0


def _anchors_np(image_height, image_width):
    aspect_ratios = [0.5, 1.0, 2.0]
    scales = [2 ** x for x in [0.0, 1.0 / 3.0, 2.0 / 3.0]]
    areas = [x ** 2 for x in [32.0, 64.0, 128.0, 256.0, 512.0]]
    strides = [2 ** i for i in range(3, 8)]
    anchors = []
    for li, stride in enumerate(strides):
        area = areas[li]
        dims = []
        for ratio in aspect_ratios:
            h = math.sqrt(area / ratio)
            w = area / h
            for s in scales:
                dims.append([s * w, s * h])
        dims = np.array(dims, dtype=np.float32)
        fh = int(math.ceil(image_height / stride))
        fw = int(math.ceil(image_width / stride))
        rx = (np.arange(fw, dtype=np.float32) + 0.5) * stride
        ry = (np.arange(fh, dtype=np.float32) + 0.5) * stride
        cx, cy = np.meshgrid(rx, ry)
        centers = np.stack([cx, cy], axis=-1)
        centers = np.tile(centers[:, :, None, :], [1, 1, 9, 1])
        d = np.tile(dims[None, None, :, :], [fh, fw, 1, 1])
        a = np.concatenate([centers, d], axis=-1).reshape(-1, 4)
        anchors.append(a)
    return np.concatenate(anchors, axis=0)


def _decode_kernel(bp_ref, an_ref, out_ref):
    t = bp_ref[0]
    a = an_ref[...]
    tx = t[0:1, :] * 0.1
    ty = t[1:2, :] * 0.1
    tw = t[2:3, :] * 0.2
    th = t[3:4, :] * 0.2
    cx = tx * a[2:3, :] + a[0:1, :]
    cy = ty * a[3:4, :] + a[1:2, :]
    w = jnp.exp(tw) * a[2:3, :]
    h = jnp.exp(th) * a[3:4, :]
    out_ref[0, 0:1, :] = cx - w / 2.0
    out_ref[0, 1:2, :] = cy - h / 2.0
    out_ref[0, 2:3, :] = cx + w / 2.0
    out_ref[0, 3:4, :] = cy + h / 2.0


def _lane_tri():
    r = jax.lax.broadcasted_iota(jnp.int32, (128, 128), 0)
    c = jax.lax.broadcasted_iota(jnp.int32, (128, 128), 1)
    return (r < c).astype(jnp.float32)


def _row_tri(n):
    r = jax.lax.broadcasted_iota(jnp.int32, (n, n), 0)
    c = jax.lax.broadcasted_iota(jnp.int32, (n, n), 1)
    return (c < r).astype(jnp.float32)


def _excl_prefix(mask_f, ltri, rtri):
    lane_p = jnp.dot(mask_f, ltri, preferred_element_type=jnp.float32)
    row_tot = jnp.sum(mask_f, axis=1, keepdims=True)
    row_base = jnp.dot(rtri, row_tot, preferred_element_type=jnp.float32)
    return lane_p + row_base


def _radix_kth_largest(key, k, signed):
    if signed:
        cnt0 = jnp.sum((key >= 0).astype(jnp.int32))
        base0 = jnp.where(cnt0 >= k, jnp.int32(0), jnp.int32(-2147483648))
    else:
        base0 = jnp.int32(0)

    def body(b, base):
        cand = base | (jnp.int32(1) << (30 - b))
        cnt = jnp.sum((key >= cand).astype(jnp.int32))
        return jnp.where(cnt >= k, cand, base)

    return jax.lax.fori_loop(0, 31, body, base0)


def _select_mask(key, thr, k, ltri, rtri):
    gt = key > thr
    eq = key == thr
    g = jnp.sum(gt.astype(jnp.int32))
    need_f = (k - g).astype(jnp.float32)
    eq_rank = _excl_prefix(eq.astype(jnp.float32), ltri, rtri)
    return gt | (eq & (eq_rank < need_f))


def _compact(slot_masked, rows, chunk_rows, n_slots, vrows):
    p = len(vrows)
    assert rows % chunk_rows == 0
    n_ch = rows // chunk_rows
    cw = chunk_rows * 128
    iota_s = jax.lax.broadcasted_iota(jnp.int32, (n_slots, 1), 0).astype(jnp.float32)

    colacc = jnp.zeros((n_slots, p), jnp.float32)
    rowacc = jnp.zeros((p, n_slots), jnp.float32)
    for ch in range(n_ch):
        r0 = ch * chunk_rows
        sl = slot_masked[r0:r0 + chunk_rows, :].reshape(1, cw)
        oh = (sl == iota_s).astype(jnp.float32)
        vs = [v[r0:r0 + chunk_rows, :].reshape(1, cw) for v in vrows]
        vm = jnp.concatenate(vs, axis=0)
        colacc = colacc + jax.lax.dot_general(
            oh, vm, (((1,), (1,)), ((), ())), precision=jax.lax.Precision.HIGHEST,
            preferred_element_type=jnp.float32)
        rowacc = rowacc + jax.lax.dot_general(
            vm, oh, (((1,), (1,)), ((), ())), precision=jax.lax.Precision.HIGHEST,
            preferred_element_type=jnp.float32)
    return colacc, rowacc


def _nms_kernel(lg_ref, bx_ref, os_ref, ob_ref):
    ltri = _lane_tri()
    rtri = _row_tri(_R)
    lg = lg_ref[0, 0, :].reshape(_R, 128)
    sig = jax.nn.sigmoid(lg)
    key = jax.lax.bitcast_convert_type(sig, jnp.int32)
    thr = _radix_kth_largest(key, _K, signed=False)
    sel = _select_mask(key, thr, _K, ltri, rtri)
    slot = _excl_prefix(sel.astype(jnp.float32), ltri, rtri)
    slot_m = jnp.where(sel, slot, -1.0)

    fe = (jax.lax.broadcasted_iota(jnp.int32, (_R, 128), 0) * 128
          + jax.lax.broadcasted_iota(jnp.int32, (_R, 128), 1)).astype(jnp.float32)
    bx = bx_ref[0]
    planes = [sig,
              bx[0, :].reshape(_R, 128), bx[1, :].reshape(_R, 128),
              bx[2, :].reshape(_R, 128), bx[3, :].reshape(_R, 128),
              fe]
    col, row = _compact(slot_m, _R, 16, _S, planes)
    sig_c, sig_r = col[:, 0:1], row[0:1, :]
    x1c, y1c, x2c, y2c = col[:, 1:2], col[:, 2:3], col[:, 3:4], col[:, 4:5]
    x1r, y1r, x2r, y2r = row[1:2, :], row[2:3, :], row[3:4, :], row[4:5, :]
    fec, fer = col[:, 5:6], row[5:6, :]

    area_c = jnp.maximum(x2c - x1c, 0.0) * jnp.maximum(y2c - y1c, 0.0)
    area_r = jnp.maximum(x2r - x1r, 0.0) * jnp.maximum(y2r - y1r, 0.0)
    ix1 = jnp.maximum(x1c, x1r)
    iy1 = jnp.maximum(y1c, y1r)
    ix2 = jnp.minimum(x2c, x2r)
    iy2 = jnp.minimum(y2c, y2r)
    inter = jnp.maximum(ix2 - ix1, 0.0) * jnp.maximum(iy2 - iy1, 0.0)
    union = area_c + area_r - inter
    iou = jnp.where(union > 0.0, inter / union, 0.0)

    prec = (sig_c > sig_r) | ((sig_c == sig_r) & (fec < fer))
    m = (prec & (iou > _IOU_THRESH)).astype(jnp.float32)
    valid = (sig_r > _CONF_THRESH).astype(jnp.float32)

    def cond(state):
        return state[1]

    def body(state):
        keep, _ = state
        supp = jnp.dot(keep, m, preferred_element_type=jnp.float32)
        new = valid * (supp == 0.0).astype(jnp.float32)
        changed = jnp.any(new != keep)
        return (new, changed)

    keep, _ = jax.lax.while_loop(cond, body, (valid, True))
    os_ref[0] = jnp.where(keep > 0.0, sig_r, _NEG)
    ob_ref[0] = row[1:5, :]


def _merge_kernel(sc_ref, bx_ref, ob_ref, os_ref, oc_ref, nv_ref):
    ltri = _lane_tri()
    rtri = _row_tri(_R3)
    s2 = sc_ref[0, 0, :].reshape(_R3, 128)
    u = jax.lax.bitcast_convert_type(s2, jnp.int32)
    key = u ^ ((u >> 31) & jnp.int32(0x7FFFFFFF))
    thr = _radix_kth_largest(key, _K3, signed=True)
    sel = _select_mask(key, thr, _K3, ltri, rtri)
    slot = _excl_prefix(sel.astype(jnp.float32), ltri, rtri)
    slot_m = jnp.where(sel, slot, -1.0)

    fe = (jax.lax.broadcasted_iota(jnp.int32, (_R3, 128), 0) * 128
          + jax.lax.broadcasted_iota(jnp.int32, (_R3, 128), 1))
    cls = (fe // _S).astype(jnp.float32)
    bx = bx_ref[0]
    planes = [s2,
              bx[0, :].reshape(_R3, 128), bx[1, :].reshape(_R3, 128),
              bx[2, :].reshape(_R3, 128), bx[3, :].reshape(_R3, 128),
              cls]
    col, row = _compact(slot_m, _R3, 8, _K3, planes)
    sig_c, sig_r = col[:, 0:1], row[0:1, :]
    mi = jax.lax.broadcasted_iota(jnp.int32, (_K3, 1), 0)
    mj = jax.lax.broadcasted_iota(jnp.int32, (1, _K3), 1)
    prec = (sig_c > sig_r) | ((sig_c == sig_r) & (mi < mj))
    rank = jnp.sum(prec.astype(jnp.float32), axis=0, keepdims=True)
    oh2 = (rank == jax.lax.broadcasted_iota(jnp.int32, (_K3, 1), 0)
           .astype(jnp.float32)).astype(jnp.float32)
    sorted_pl = jnp.dot(oh2, col, precision=jax.lax.Precision.HIGHEST,
                        preferred_element_type=jnp.float32)
    score_s = sorted_pl[:, 0:1]
    vmask = score_s != _NEG
    ob_ref[0] = jnp.where(vmask, sorted_pl[:, 1:5], 0.0)
    os_ref[0] = jnp.where(vmask, score_s, 0.0)
    oc_ref[0] = jnp.where(vmask, sorted_pl[:, 5:6], 0.0)
    nfin = jnp.sum((s2 != _NEG).astype(jnp.float32))
    nv_ref[0] = jnp.zeros((1, 128), jnp.float32) + jnp.minimum(nfin, float(_MAX_TOTAL))


def kernel(images, predictions):
    B = predictions.shape[0]
    H, W = images.shape[1], images.shape[2]
    A = predictions.shape[1]
    anc = _anchors_np(H, W)
    anc_t = np.zeros((4, _AP), np.float32)
    anc_t[:, :A] = anc.T
    anc_t = jnp.asarray(anc_t)

    pred_t = jnp.transpose(predictions, (0, 2, 1))
    box_p = jnp.pad(pred_t[:, :4, :], ((0, 0), (0, 0), (0, _AP - A)))
    logits = jnp.pad(pred_t[:, 4:, :], ((0, 0), (0, 0), (0, _AP - A)),
                     constant_values=-1e30)
    logits = logits.reshape(B * _NUM_CLASSES, 1, _AP)

    corners = pl.pallas_call(
        _decode_kernel,
        out_shape=jax.ShapeDtypeStruct((B, 4, _AP), jnp.float32),
        grid=(B,),
        in_specs=[pl.BlockSpec((1, 4, _AP), lambda b: (b, 0, 0)),
                  pl.BlockSpec((4, _AP), lambda b: (0, 0))],
        out_specs=pl.BlockSpec((1, 4, _AP), lambda b: (b, 0, 0)),
    )(box_p, anc_t)

    nC = _NUM_CLASSES
    cls_scores, cls_boxes = pl.pallas_call(
        _nms_kernel,
        out_shape=(jax.ShapeDtypeStruct((B * nC, 1, _S), jnp.float32),
                   jax.ShapeDtypeStruct((B * nC, 4, _S), jnp.float32)),
        grid=(B * nC,),
        in_specs=[pl.BlockSpec((1, 1, _AP), lambda i: (i, 0, 0)),
                  pl.BlockSpec((1, 4, _AP), lambda i: (i // nC, 0, 0))],
        out_specs=(pl.BlockSpec((1, 1, _S), lambda i: (i, 0, 0)),
                   pl.BlockSpec((1, 4, _S), lambda i: (i, 0, 0))),
    )(logits, corners)

    scores_m = cls_scores.reshape(B, 1, _M3)
    boxes_m = (cls_boxes.reshape(B, nC, 4, _S)
               .transpose(0, 2, 1, 3).reshape(B, 4, _M3))

    ob, osc, ocl, nv = pl.pallas_call(
        _merge_kernel,
        out_shape=(jax.ShapeDtypeStruct((B, _K3, 4), jnp.float32),
                   jax.ShapeDtypeStruct((B, _K3, 1), jnp.float32),
                   jax.ShapeDtypeStruct((B, _K3, 1), jnp.float32),
                   jax.ShapeDtypeStruct((B, 1, 128), jnp.float32)),
        grid=(B,),
        in_specs=[pl.BlockSpec((1, 1, _M3), lambda b: (b, 0, 0)),
                  pl.BlockSpec((1, 4, _M3), lambda b: (b, 0, 0))],
        out_specs=(pl.BlockSpec((1, _K3, 4), lambda b: (b, 0, 0)),
                   pl.BlockSpec((1, _K3, 1), lambda b: (b, 0, 0)),
                   pl.BlockSpec((1, _K3, 1), lambda b: (b, 0, 0)),
                   pl.BlockSpec((1, 1, 128), lambda b: (b, 0, 0))),
    )(scores_m, boxes_m)

    out_boxes = ob[:, :_MAX_TOTAL, :]
    out_scores = osc[:, :_MAX_TOTAL, 0]
    out_classes = ocl[:, :_MAX_TOTAL, 0]
    num_valid = nv[:, 0, 0].astype(jnp.int32)
    return out_boxes, out_scores, out_classes, num_valid


# speedup vs baseline: 4.8642x; 3.6642x over previous
import math

import jax
import jax.numpy as jnp
import numpy as np
from jax.experimental import pallas as pl
from jax.experimental.pallas import tpu as pltpu

_NUM_CLASSES = 18
_CONF_THRESH = 0.05
_IOU_THRESH = 0.5
_K = 500
_S = 512
_K3 = 1024
_MAX_TOTAL = 1000
_A = 49104
_AP = 49152
_R = 384
_M3 = _NUM_CLASSES * _S
_R3 = 72
_NEG = -1.0


def _anchors_np(image_height, image_width):
    aspect_ratios = [0.5, 1.0, 2.0]
    scales = [2 ** x for x in [0.0, 1.0 / 3.0, 2.0 / 3.0]]
    areas = [x ** 2 for x in [32.0, 64.0, 128.0, 256.0, 512.0]]
    strides = [2 ** i for i in range(3, 8)]
    anchors = []
    for li, stride in enumerate(strides):
        area = areas[li]
        dims = []
        for ratio in aspect_ratios:
            h = math.sqrt(area / ratio)
            w = area / h
            for s in scales:
                dims.append([s * w, s * h])
        dims = np.array(dims, dtype=np.float32)
        fh = int(math.ceil(image_height / stride))
        fw = int(math.ceil(image_width / stride))
        rx = (np.arange(fw, dtype=np.float32) + 0.5) * stride
        ry = (np.arange(fh, dtype=np.float32) + 0.5) * stride
        cx, cy = np.meshgrid(rx, ry)
        centers = np.stack([cx, cy], axis=-1)
        centers = np.tile(centers[:, :, None, :], [1, 1, 9, 1])
        d = np.tile(dims[None, None, :, :], [fh, fw, 1, 1])
        a = np.concatenate([centers, d], axis=-1).reshape(-1, 4)
        anchors.append(a)
    return np.concatenate(anchors, axis=0)


def _decode_kernel(bp_ref, an_ref, out_ref):
    t = bp_ref[0]
    a = an_ref[...]
    tx = t[0:1, :] * 0.1
    ty = t[1:2, :] * 0.1
    tw = t[2:3, :] * 0.2
    th = t[3:4, :] * 0.2
    cx = tx * a[2:3, :] + a[0:1, :]
    cy = ty * a[3:4, :] + a[1:2, :]
    w = jnp.exp(tw) * a[2:3, :]
    h = jnp.exp(th) * a[3:4, :]
    out_ref[0, 0:1, :] = cx - w / 2.0
    out_ref[0, 1:2, :] = cy - h / 2.0
    out_ref[0, 2:3, :] = cx + w / 2.0
    out_ref[0, 3:4, :] = cy + h / 2.0


def _lane_tri():
    r = jax.lax.broadcasted_iota(jnp.int32, (128, 128), 0)
    c = jax.lax.broadcasted_iota(jnp.int32, (128, 128), 1)
    return (r < c).astype(jnp.float32)


def _row_tri(n):
    r = jax.lax.broadcasted_iota(jnp.int32, (n, n), 0)
    c = jax.lax.broadcasted_iota(jnp.int32, (n, n), 1)
    return (c < r).astype(jnp.float32)


def _excl_prefix(mask_f, ltri, rtri):
    lane_p = jnp.dot(mask_f, ltri, preferred_element_type=jnp.float32)
    row_tot = jnp.sum(mask_f, axis=1, keepdims=True)
    row_base = jnp.dot(rtri, row_tot, preferred_element_type=jnp.float32)
    return lane_p + row_base


def _radix_kth_largest(key, k, signed):
    if signed:
        cnt0 = jnp.sum((key >= 0).astype(jnp.int32))
        base0 = jnp.where(cnt0 >= k, jnp.int32(0), jnp.int32(-2147483648))
    else:
        base0 = jnp.int32(0)

    def body(b, base):
        cand = base | (jnp.int32(1) << (30 - b))
        cnt = jnp.sum((key >= cand).astype(jnp.int32))
        return jnp.where(cnt >= k, cand, base)

    return jax.lax.fori_loop(0, 31, body, base0)


def _select_mask(key, thr, k, ltri, rtri):
    gt = key > thr
    eq = key == thr
    g = jnp.sum(gt.astype(jnp.int32))
    need_f = (k - g).astype(jnp.float32)
    eq_rank = _excl_prefix(eq.astype(jnp.float32), ltri, rtri)
    return gt | (eq & (eq_rank < need_f))


def _compact(slot_masked, rows, chunk_rows, n_slots, vrows):
    p = len(vrows)
    assert rows % chunk_rows == 0
    n_ch = rows // chunk_rows
    cw = chunk_rows * 128
    iota_s = jax.lax.broadcasted_iota(jnp.int32, (n_slots, 1), 0).astype(jnp.float32)

    colacc = jnp.zeros((n_slots, p), jnp.float32)
    for ch in range(n_ch):
        r0 = ch * chunk_rows
        sl = slot_masked[r0:r0 + chunk_rows, :].reshape(1, cw)
        oh = (sl == iota_s).astype(jnp.bfloat16)
        vs = [v[r0:r0 + chunk_rows, :].reshape(1, cw) for v in vrows]
        vm = jnp.concatenate(vs, axis=0)
        v1 = vm.astype(jnp.bfloat16)
        r1 = vm - v1.astype(jnp.float32)
        v2 = r1.astype(jnp.bfloat16)
        v3 = (r1 - v2.astype(jnp.float32)).astype(jnp.bfloat16)
        for vpart in (v1, v2, v3):
            colacc = colacc + jax.lax.dot_general(
                oh, vpart, (((1,), (1,)), ((), ())),
                preferred_element_type=jnp.float32)
    return colacc, jnp.transpose(colacc)


def _nms_kernel(lg_ref, bx_ref, os_ref, ob_ref):
    ltri = _lane_tri()
    rtri = _row_tri(_R)
    lg = lg_ref[0, 0, :].reshape(_R, 128)
    sig = jax.nn.sigmoid(lg)
    key = jax.lax.bitcast_convert_type(sig, jnp.int32)
    thr = _radix_kth_largest(key, _K, signed=False)
    sel = _select_mask(key, thr, _K, ltri, rtri)
    slot = _excl_prefix(sel.astype(jnp.float32), ltri, rtri)
    slot_m = jnp.where(sel, slot, -1.0)

    fe = (jax.lax.broadcasted_iota(jnp.int32, (_R, 128), 0) * 128
          + jax.lax.broadcasted_iota(jnp.int32, (_R, 128), 1)).astype(jnp.float32)
    bx = bx_ref[0]
    planes = [sig,
              bx[0, :].reshape(_R, 128), bx[1, :].reshape(_R, 128),
              bx[2, :].reshape(_R, 128), bx[3, :].reshape(_R, 128),
              fe]
    col, row = _compact(slot_m, _R, 16, _S, planes)
    sig_c, sig_r = col[:, 0:1], row[0:1, :]
    x1c, y1c, x2c, y2c = col[:, 1:2], col[:, 2:3], col[:, 3:4], col[:, 4:5]
    x1r, y1r, x2r, y2r = row[1:2, :], row[2:3, :], row[3:4, :], row[4:5, :]
    fec, fer = col[:, 5:6], row[5:6, :]

    area_c = jnp.maximum(x2c - x1c, 0.0) * jnp.maximum(y2c - y1c, 0.0)
    area_r = jnp.maximum(x2r - x1r, 0.0) * jnp.maximum(y2r - y1r, 0.0)
    ix1 = jnp.maximum(x1c, x1r)
    iy1 = jnp.maximum(y1c, y1r)
    ix2 = jnp.minimum(x2c, x2r)
    iy2 = jnp.minimum(y2c, y2r)
    inter = jnp.maximum(ix2 - ix1, 0.0) * jnp.maximum(iy2 - iy1, 0.0)
    union = area_c + area_r - inter
    iou = jnp.where(union > 0.0, inter / union, 0.0)

    prec = (sig_c > sig_r) | ((sig_c == sig_r) & (fec < fer))
    m = (prec & (iou > _IOU_THRESH)).astype(jnp.float32)
    valid = (sig_r > _CONF_THRESH).astype(jnp.float32)

    def cond(state):
        return state[1]

    def body(state):
        keep, _ = state
        supp = jnp.dot(keep, m, preferred_element_type=jnp.float32)
        new = valid * (supp == 0.0).astype(jnp.float32)
        changed = jnp.any(new != keep)
        return (new, changed)

    keep, _ = jax.lax.while_loop(cond, body, (valid, True))
    os_ref[0] = jnp.where(keep > 0.0, sig_r, _NEG)
    ob_ref[0] = row[1:5, :]


def _merge_kernel(sc_ref, bx_ref, ob_ref, os_ref, oc_ref, nv_ref):
    ltri = _lane_tri()
    rtri = _row_tri(_R3)
    s2 = sc_ref[0, 0, :].reshape(_R3, 128)
    u = jax.lax.bitcast_convert_type(s2, jnp.int32)
    key = u ^ ((u >> 31) & jnp.int32(0x7FFFFFFF))
    thr = _radix_kth_largest(key, _K3, signed=True)
    sel = _select_mask(key, thr, _K3, ltri, rtri)
    slot = _excl_prefix(sel.astype(jnp.float32), ltri, rtri)
    slot_m = jnp.where(sel, slot, -1.0)

    fe = (jax.lax.broadcasted_iota(jnp.int32, (_R3, 128), 0) * 128
          + jax.lax.broadcasted_iota(jnp.int32, (_R3, 128), 1))
    cls = (fe // _S).astype(jnp.float32)
    bx = bx_ref[0]
    planes = [s2,
              bx[0, :].reshape(_R3, 128), bx[1, :].reshape(_R3, 128),
              bx[2, :].reshape(_R3, 128), bx[3, :].reshape(_R3, 128),
              cls]
    col, row = _compact(slot_m, _R3, 8, _K3, planes)
    sig_c, sig_r = col[:, 0:1], row[0:1, :]
    mi = jax.lax.broadcasted_iota(jnp.int32, (_K3, 1), 0)
    mj = jax.lax.broadcasted_iota(jnp.int32, (1, _K3), 1)
    prec = (sig_c > sig_r) | ((sig_c == sig_r) & (mi < mj))
    rank = jnp.sum(prec.astype(jnp.float32), axis=0, keepdims=True)
    oh2 = (rank == jax.lax.broadcasted_iota(jnp.int32, (_K3, 1), 0)
           .astype(jnp.float32)).astype(jnp.float32)
    sorted_pl = jnp.dot(oh2, col, precision=jax.lax.Precision.HIGHEST,
                        preferred_element_type=jnp.float32)
    score_s = sorted_pl[:, 0:1]
    vmask = score_s != _NEG
    ob_ref[0] = jnp.where(vmask, sorted_pl[:, 1:5], 0.0)
    os_ref[0] = jnp.where(vmask, score_s, 0.0)
    oc_ref[0] = jnp.where(vmask, sorted_pl[:, 5:6], 0.0)
    nfin = jnp.sum((s2 != _NEG).astype(jnp.float32))
    nv_ref[0] = jnp.zeros((1, 128), jnp.float32) + jnp.minimum(nfin, float(_MAX_TOTAL))


def kernel(images, predictions):
    B = predictions.shape[0]
    H, W = images.shape[1], images.shape[2]
    A = predictions.shape[1]
    anc = _anchors_np(H, W)
    anc_t = np.zeros((4, _AP), np.float32)
    anc_t[:, :A] = anc.T
    anc_t = jnp.asarray(anc_t)

    pred_t = jnp.transpose(predictions, (0, 2, 1))
    box_p = jnp.pad(pred_t[:, :4, :], ((0, 0), (0, 0), (0, _AP - A)))
    logits = jnp.pad(pred_t[:, 4:, :], ((0, 0), (0, 0), (0, _AP - A)),
                     constant_values=-1e30)
    logits = logits.reshape(B * _NUM_CLASSES, 1, _AP)

    corners = pl.pallas_call(
        _decode_kernel,
        out_shape=jax.ShapeDtypeStruct((B, 4, _AP), jnp.float32),
        grid=(B,),
        in_specs=[pl.BlockSpec((1, 4, _AP), lambda b: (b, 0, 0)),
                  pl.BlockSpec((4, _AP), lambda b: (0, 0))],
        out_specs=pl.BlockSpec((1, 4, _AP), lambda b: (b, 0, 0)),
    )(box_p, anc_t)

    nC = _NUM_CLASSES
    cls_scores, cls_boxes = pl.pallas_call(
        _nms_kernel,
        out_shape=(jax.ShapeDtypeStruct((B * nC, 1, _S), jnp.float32),
                   jax.ShapeDtypeStruct((B * nC, 4, _S), jnp.float32)),
        grid=(B * nC,),
        in_specs=[pl.BlockSpec((1, 1, _AP), lambda i: (i, 0, 0)),
                  pl.BlockSpec((1, 4, _AP), lambda i: (i // nC, 0, 0))],
        out_specs=(pl.BlockSpec((1, 1, _S), lambda i: (i, 0, 0)),
                   pl.BlockSpec((1, 4, _S), lambda i: (i, 0, 0))),
    )(logits, corners)

    scores_m = cls_scores.reshape(B, 1, _M3)
    boxes_m = (cls_boxes.reshape(B, nC, 4, _S)
               .transpose(0, 2, 1, 3).reshape(B, 4, _M3))

    ob, osc, ocl, nv = pl.pallas_call(
        _merge_kernel,
        out_shape=(jax.ShapeDtypeStruct((B, _K3, 4), jnp.float32),
                   jax.ShapeDtypeStruct((B, _K3, 1), jnp.float32),
                   jax.ShapeDtypeStruct((B, _K3, 1), jnp.float32),
                   jax.ShapeDtypeStruct((B, 1, 128), jnp.float32)),
        grid=(B,),
        in_specs=[pl.BlockSpec((1, 1, _M3), lambda b: (b, 0, 0)),
                  pl.BlockSpec((1, 4, _M3), lambda b: (b, 0, 0))],
        out_specs=(pl.BlockSpec((1, _K3, 4), lambda b: (b, 0, 0)),
                   pl.BlockSpec((1, _K3, 1), lambda b: (b, 0, 0)),
                   pl.BlockSpec((1, _K3, 1), lambda b: (b, 0, 0)),
                   pl.BlockSpec((1, 1, 128), lambda b: (b, 0, 0))),
    )(scores_m, boxes_m)

    out_boxes = ob[:, :_MAX_TOTAL, :]
    out_scores = osc[:, :_MAX_TOTAL, 0]
    out_classes = ocl[:, :_MAX_TOTAL, 0]
    num_valid = nv[:, 0, 0].astype(jnp.int32)
    return out_boxes, out_scores, out_classes, num_valid


# speedup vs baseline: 8.8945x; 1.8286x over previous
import math

import jax
import jax.numpy as jnp
import numpy as np
from jax.experimental import pallas as pl
from jax.experimental.pallas import tpu as pltpu

_NUM_CLASSES = 18
_CONF_THRESH = 0.05
_IOU_THRESH = 0.5
_K = 500
_S = 512
_K3 = 1024
_MAX_TOTAL = 1000
_A = 49104
_AP = 49152
_R = 384
_M3 = _NUM_CLASSES * _S
_R3 = 72
_NEG = -1.0


def _anchors_np(image_height, image_width):
    aspect_ratios = [0.5, 1.0, 2.0]
    scales = [2 ** x for x in [0.0, 1.0 / 3.0, 2.0 / 3.0]]
    areas = [x ** 2 for x in [32.0, 64.0, 128.0, 256.0, 512.0]]
    strides = [2 ** i for i in range(3, 8)]
    anchors = []
    for li, stride in enumerate(strides):
        area = areas[li]
        dims = []
        for ratio in aspect_ratios:
            h = math.sqrt(area / ratio)
            w = area / h
            for s in scales:
                dims.append([s * w, s * h])
        dims = np.array(dims, dtype=np.float32)
        fh = int(math.ceil(image_height / stride))
        fw = int(math.ceil(image_width / stride))
        rx = (np.arange(fw, dtype=np.float32) + 0.5) * stride
        ry = (np.arange(fh, dtype=np.float32) + 0.5) * stride
        cx, cy = np.meshgrid(rx, ry)
        centers = np.stack([cx, cy], axis=-1)
        centers = np.tile(centers[:, :, None, :], [1, 1, 9, 1])
        d = np.tile(dims[None, None, :, :], [fh, fw, 1, 1])
        a = np.concatenate([centers, d], axis=-1).reshape(-1, 4)
        anchors.append(a)
    return np.concatenate(anchors, axis=0)


def _decode_kernel(bp_ref, an_ref, out_ref):
    t = bp_ref[0]
    a = an_ref[...]
    tx = t[0:1, :] * 0.1
    ty = t[1:2, :] * 0.1
    tw = t[2:3, :] * 0.2
    th = t[3:4, :] * 0.2
    cx = tx * a[2:3, :] + a[0:1, :]
    cy = ty * a[3:4, :] + a[1:2, :]
    w = jnp.exp(tw) * a[2:3, :]
    h = jnp.exp(th) * a[3:4, :]
    out_ref[0, 0:1, :] = cx - w / 2.0
    out_ref[0, 1:2, :] = cy - h / 2.0
    out_ref[0, 2:3, :] = cx + w / 2.0
    out_ref[0, 3:4, :] = cy + h / 2.0


def _lane_tri():
    r = jax.lax.broadcasted_iota(jnp.int32, (128, 128), 0)
    c = jax.lax.broadcasted_iota(jnp.int32, (128, 128), 1)
    return (r < c).astype(jnp.float32)


def _row_tri(n):
    r = jax.lax.broadcasted_iota(jnp.int32, (n, n), 0)
    c = jax.lax.broadcasted_iota(jnp.int32, (n, n), 1)
    return (c < r).astype(jnp.float32)


def _excl_prefix(mask_f, ltri, rtri):
    lane_p = jnp.dot(mask_f, ltri, preferred_element_type=jnp.float32)
    row_tot = jnp.sum(mask_f, axis=1, keepdims=True)
    row_base = jnp.dot(rtri, row_tot, preferred_element_type=jnp.float32)
    return lane_p + row_base


def _radix_kth_largest(key, k, signed):
    if signed:
        cnt0 = jnp.sum((key >= 0).astype(jnp.int32))
        base0 = jnp.where(cnt0 >= k, jnp.int32(0), jnp.int32(-2147483648))
    else:
        base0 = jnp.int32(0)

    def body(b, base):
        cand = base | (jnp.int32(1) << (30 - b))
        cnt = jnp.sum((key >= cand).astype(jnp.int32))
        return jnp.where(cnt >= k, cand, base)

    return jax.lax.fori_loop(0, 31, body, base0)


def _select_mask(key, thr, k, ltri, rtri):
    gt = key > thr
    eq = key == thr
    g = jnp.sum(gt.astype(jnp.int32))
    need_f = (k - g).astype(jnp.float32)
    eq_rank = _excl_prefix(eq.astype(jnp.float32), ltri, rtri)
    return gt | (eq & (eq_rank < need_f))


def _compact(slot_masked, rows, chunk_rows, n_slots, vrows):
    p = len(vrows)
    assert rows % chunk_rows == 0
    n_ch = rows // chunk_rows
    cw = chunk_rows * 128
    iota_s = jax.lax.broadcasted_iota(jnp.int32, (n_slots, 1), 0).astype(jnp.int16)

    colacc = jnp.zeros((n_slots, 3 * p), jnp.float32)
    for ch in range(n_ch):
        r0 = ch * chunk_rows
        sl = slot_masked[r0:r0 + chunk_rows, :].reshape(1, cw).astype(jnp.int16)
        oh = (sl == iota_s).astype(jnp.bfloat16)
        vs = [v[r0:r0 + chunk_rows, :].reshape(1, cw) for v in vrows]
        vm = jnp.concatenate(vs, axis=0)
        v1 = vm.astype(jnp.bfloat16)
        r1 = vm - v1.astype(jnp.float32)
        v2 = r1.astype(jnp.bfloat16)
        v3 = (r1 - v2.astype(jnp.float32)).astype(jnp.bfloat16)
        vcat = jnp.concatenate([v1, v2, v3], axis=0)
        colacc = colacc + jax.lax.dot_general(
            oh, vcat, (((1,), (1,)), ((), ())),
            preferred_element_type=jnp.float32)
    col = colacc[:, 0:p] + colacc[:, p:2 * p] + colacc[:, 2 * p:3 * p]
    return col, jnp.transpose(col)


def _nms_kernel(lg_ref, bx_ref, os_ref, ob_ref):
    ltri = _lane_tri()
    rtri = _row_tri(_R)
    lg = lg_ref[0, 0, :].reshape(_R, 128)
    sig = jax.nn.sigmoid(lg)
    key = jax.lax.bitcast_convert_type(sig, jnp.int32)
    thr = _radix_kth_largest(key, _K, signed=False)
    sel = _select_mask(key, thr, _K, ltri, rtri)
    slot = _excl_prefix(sel.astype(jnp.float32), ltri, rtri)
    slot_m = jnp.where(sel, slot, -1.0)

    fe = (jax.lax.broadcasted_iota(jnp.int32, (_R, 128), 0) * 128
          + jax.lax.broadcasted_iota(jnp.int32, (_R, 128), 1)).astype(jnp.float32)
    bx = bx_ref[0]
    planes = [sig,
              bx[0, :].reshape(_R, 128), bx[1, :].reshape(_R, 128),
              bx[2, :].reshape(_R, 128), bx[3, :].reshape(_R, 128),
              fe]
    col, row = _compact(slot_m, _R, 16, _S, planes)
    sig_c, sig_r = col[:, 0:1], row[0:1, :]
    x1c, y1c, x2c, y2c = col[:, 1:2], col[:, 2:3], col[:, 3:4], col[:, 4:5]
    x1r, y1r, x2r, y2r = row[1:2, :], row[2:3, :], row[3:4, :], row[4:5, :]
    fec, fer = col[:, 5:6], row[5:6, :]

    area_c = jnp.maximum(x2c - x1c, 0.0) * jnp.maximum(y2c - y1c, 0.0)
    area_r = jnp.maximum(x2r - x1r, 0.0) * jnp.maximum(y2r - y1r, 0.0)
    ix1 = jnp.maximum(x1c, x1r)
    iy1 = jnp.maximum(y1c, y1r)
    ix2 = jnp.minimum(x2c, x2r)
    iy2 = jnp.minimum(y2c, y2r)
    inter = jnp.maximum(ix2 - ix1, 0.0) * jnp.maximum(iy2 - iy1, 0.0)
    union = area_c + area_r - inter
    iou = jnp.where(union > 0.0, inter / union, 0.0)

    prec = (sig_c > sig_r) | ((sig_c == sig_r) & (fec < fer))
    m = (prec & (iou > _IOU_THRESH)).astype(jnp.float32)
    valid = (sig_r > _CONF_THRESH).astype(jnp.float32)

    def cond(state):
        return state[1]

    def body(state):
        keep, _ = state
        s1 = jnp.dot(keep, m, preferred_element_type=jnp.float32)
        k1 = valid * (s1 == 0.0).astype(jnp.float32)
        s2 = jnp.dot(k1, m, preferred_element_type=jnp.float32)
        k2 = valid * (s2 == 0.0).astype(jnp.float32)
        changed = jnp.any(k2 != k1)
        return (k2, changed)

    keep, _ = jax.lax.while_loop(cond, body, (valid, True))
    os_ref[0] = jnp.where(keep > 0.0, sig_r, _NEG)
    ob_ref[0] = row[1:5, :]


def _merge_kernel(sc_ref, bx_ref, ob_ref, os_ref, oc_ref, nv_ref):
    ltri = _lane_tri()
    rtri = _row_tri(_R3)
    s2 = sc_ref[0, 0, :].reshape(_R3, 128)
    u = jax.lax.bitcast_convert_type(s2, jnp.int32)
    key = u ^ ((u >> 31) & jnp.int32(0x7FFFFFFF))
    thr = _radix_kth_largest(key, _K3, signed=True)
    sel = _select_mask(key, thr, _K3, ltri, rtri)
    slot = _excl_prefix(sel.astype(jnp.float32), ltri, rtri)
    slot_m = jnp.where(sel, slot, -1.0)

    fe = (jax.lax.broadcasted_iota(jnp.int32, (_R3, 128), 0) * 128
          + jax.lax.broadcasted_iota(jnp.int32, (_R3, 128), 1))
    cls = (fe // _S).astype(jnp.float32)
    bx = bx_ref[0]
    planes = [s2,
              bx[0, :].reshape(_R3, 128), bx[1, :].reshape(_R3, 128),
              bx[2, :].reshape(_R3, 128), bx[3, :].reshape(_R3, 128),
              cls]
    col, row = _compact(slot_m, _R3, 8, _K3, planes)
    sig_c, sig_r = col[:, 0:1], row[0:1, :]
    mi = jax.lax.broadcasted_iota(jnp.int32, (_K3, 1), 0)
    mj = jax.lax.broadcasted_iota(jnp.int32, (1, _K3), 1)
    prec = (sig_c > sig_r) | ((sig_c == sig_r) & (mi < mj))
    rank = jnp.sum(prec.astype(jnp.float32), axis=0, keepdims=True)
    oh2 = (rank == jax.lax.broadcasted_iota(jnp.int32, (_K3, 1), 0)
           .astype(jnp.float32)).astype(jnp.bfloat16)
    c1 = col.astype(jnp.bfloat16)
    cr1 = col - c1.astype(jnp.float32)
    c2 = cr1.astype(jnp.bfloat16)
    c3 = (cr1 - c2.astype(jnp.float32)).astype(jnp.bfloat16)
    ccat = jnp.concatenate([c1, c2, c3], axis=1)
    sp = jnp.dot(oh2, ccat, preferred_element_type=jnp.float32)
    np_ = col.shape[1]
    sorted_pl = sp[:, 0:np_] + sp[:, np_:2 * np_] + sp[:, 2 * np_:3 * np_]
    score_s = sorted_pl[:, 0:1]
    vmask = score_s != _NEG
    ob_ref[0] = jnp.where(vmask, sorted_pl[:, 1:5], 0.0)
    os_ref[0] = jnp.where(vmask, score_s, 0.0)
    oc_ref[0] = jnp.where(vmask, sorted_pl[:, 5:6], 0.0)
    nfin = jnp.sum((s2 != _NEG).astype(jnp.float32))
    nv_ref[0] = jnp.zeros((1, 128), jnp.float32) + jnp.minimum(nfin, float(_MAX_TOTAL))


def kernel(images, predictions):
    B = predictions.shape[0]
    H, W = images.shape[1], images.shape[2]
    A = predictions.shape[1]
    anc = _anchors_np(H, W)
    anc_t = np.zeros((4, _AP), np.float32)
    anc_t[:, :A] = anc.T
    anc_t = jnp.asarray(anc_t)

    pred_t = jnp.transpose(predictions, (0, 2, 1))
    box_p = jnp.pad(pred_t[:, :4, :], ((0, 0), (0, 0), (0, _AP - A)))
    logits = jnp.pad(pred_t[:, 4:, :], ((0, 0), (0, 0), (0, _AP - A)),
                     constant_values=-1e30)
    logits = logits.reshape(B * _NUM_CLASSES, 1, _AP)

    corners = pl.pallas_call(
        _decode_kernel,
        out_shape=jax.ShapeDtypeStruct((B, 4, _AP), jnp.float32),
        grid=(B,),
        in_specs=[pl.BlockSpec((1, 4, _AP), lambda b: (b, 0, 0)),
                  pl.BlockSpec((4, _AP), lambda b: (0, 0))],
        out_specs=pl.BlockSpec((1, 4, _AP), lambda b: (b, 0, 0)),
    )(box_p, anc_t)

    nC = _NUM_CLASSES
    cls_scores, cls_boxes = pl.pallas_call(
        _nms_kernel,
        out_shape=(jax.ShapeDtypeStruct((B * nC, 1, _S), jnp.float32),
                   jax.ShapeDtypeStruct((B * nC, 4, _S), jnp.float32)),
        grid=(B * nC,),
        in_specs=[pl.BlockSpec((1, 1, _AP), lambda i: (i, 0, 0)),
                  pl.BlockSpec((1, 4, _AP), lambda i: (i // nC, 0, 0))],
        out_specs=(pl.BlockSpec((1, 1, _S), lambda i: (i, 0, 0)),
                   pl.BlockSpec((1, 4, _S), lambda i: (i, 0, 0))),
    )(logits, corners)

    scores_m = cls_scores.reshape(B, 1, _M3)
    boxes_m = (cls_boxes.reshape(B, nC, 4, _S)
               .transpose(0, 2, 1, 3).reshape(B, 4, _M3))

    ob, osc, ocl, nv = pl.pallas_call(
        _merge_kernel,
        out_shape=(jax.ShapeDtypeStruct((B, _K3, 4), jnp.float32),
                   jax.ShapeDtypeStruct((B, _K3, 1), jnp.float32),
                   jax.ShapeDtypeStruct((B, _K3, 1), jnp.float32),
                   jax.ShapeDtypeStruct((B, 1, 128), jnp.float32)),
        grid=(B,),
        in_specs=[pl.BlockSpec((1, 1, _M3), lambda b: (b, 0, 0)),
                  pl.BlockSpec((1, 4, _M3), lambda b: (b, 0, 0))],
        out_specs=(pl.BlockSpec((1, _K3, 4), lambda b: (b, 0, 0)),
                   pl.BlockSpec((1, _K3, 1), lambda b: (b, 0, 0)),
                   pl.BlockSpec((1, _K3, 1), lambda b: (b, 0, 0)),
                   pl.BlockSpec((1, 1, 128), lambda b: (b, 0, 0))),
    )(scores_m, boxes_m)

    out_boxes = ob[:, :_MAX_TOTAL, :]
    out_scores = osc[:, :_MAX_TOTAL, 0]
    out_classes = ocl[:, :_MAX_TOTAL, 0]
    num_valid = nv[:, 0, 0].astype(jnp.int32)
    return out_boxes, out_scores, out_classes, num_valid


# speedup vs baseline: 10.3799x; 1.1670x over previous
import math

import jax
import jax.numpy as jnp
import numpy as np
from jax.experimental import pallas as pl
from jax.experimental.pallas import tpu as pltpu

_NUM_CLASSES = 18
_CONF_THRESH = 0.05
_IOU_THRESH = 0.5
_K = 500
_S = 512
_K3 = 1024
_MAX_TOTAL = 1000
_A = 49104
_AP = 49152
_R = 384
_M3 = _NUM_CLASSES * _S
_R3 = 72
_NEG = -1.0


def _anchors_np(image_height, image_width):
    aspect_ratios = [0.5, 1.0, 2.0]
    scales = [2 ** x for x in [0.0, 1.0 / 3.0, 2.0 / 3.0]]
    areas = [x ** 2 for x in [32.0, 64.0, 128.0, 256.0, 512.0]]
    strides = [2 ** i for i in range(3, 8)]
    anchors = []
    for li, stride in enumerate(strides):
        area = areas[li]
        dims = []
        for ratio in aspect_ratios:
            h = math.sqrt(area / ratio)
            w = area / h
            for s in scales:
                dims.append([s * w, s * h])
        dims = np.array(dims, dtype=np.float32)
        fh = int(math.ceil(image_height / stride))
        fw = int(math.ceil(image_width / stride))
        rx = (np.arange(fw, dtype=np.float32) + 0.5) * stride
        ry = (np.arange(fh, dtype=np.float32) + 0.5) * stride
        cx, cy = np.meshgrid(rx, ry)
        centers = np.stack([cx, cy], axis=-1)
        centers = np.tile(centers[:, :, None, :], [1, 1, 9, 1])
        d = np.tile(dims[None, None, :, :], [fh, fw, 1, 1])
        a = np.concatenate([centers, d], axis=-1).reshape(-1, 4)
        anchors.append(a)
    return np.concatenate(anchors, axis=0)


def _decode_kernel(bp_ref, an_ref, out_ref):
    t = bp_ref[0]
    a = an_ref[...]
    tx = t[0:1, :] * 0.1
    ty = t[1:2, :] * 0.1
    tw = t[2:3, :] * 0.2
    th = t[3:4, :] * 0.2
    cx = tx * a[2:3, :] + a[0:1, :]
    cy = ty * a[3:4, :] + a[1:2, :]
    w = jnp.exp(tw) * a[2:3, :]
    h = jnp.exp(th) * a[3:4, :]
    out_ref[0, 0:1, :] = cx - w / 2.0
    out_ref[0, 1:2, :] = cy - h / 2.0
    out_ref[0, 2:3, :] = cx + w / 2.0
    out_ref[0, 3:4, :] = cy + h / 2.0


def _lane_tri():
    r = jax.lax.broadcasted_iota(jnp.int32, (128, 128), 0)
    c = jax.lax.broadcasted_iota(jnp.int32, (128, 128), 1)
    return (r < c).astype(jnp.float32)


def _row_tri(n):
    r = jax.lax.broadcasted_iota(jnp.int32, (n, n), 0)
    c = jax.lax.broadcasted_iota(jnp.int32, (n, n), 1)
    return (c < r).astype(jnp.float32)


def _excl_prefix(mask_f, ltri, rtri):
    lane_p = jnp.dot(mask_f, ltri, preferred_element_type=jnp.float32)
    row_tot = jnp.sum(mask_f, axis=1, keepdims=True)
    row_base = jnp.dot(rtri, row_tot, preferred_element_type=jnp.float32)
    return lane_p + row_base


def _radix_kth_largest(key, k, signed):
    if signed:
        cnt0 = jnp.sum((key >= 0).astype(jnp.int32))
        base0 = jnp.where(cnt0 >= k, jnp.int32(0), jnp.int32(-2147483648))
    else:
        base0 = jnp.int32(0)

    def body(b, base):
        cand = base | (jnp.int32(1) << (30 - b))
        cnt = jnp.sum((key >= cand).astype(jnp.int32))
        return jnp.where(cnt >= k, cand, base)

    return jax.lax.fori_loop(0, 31, body, base0)


def _select_mask(key, thr, k, ltri, rtri):
    gt = key > thr
    eq = key == thr
    g = jnp.sum(gt.astype(jnp.int32))
    need_f = (k - g).astype(jnp.float32)
    eq_rank = _excl_prefix(eq.astype(jnp.float32), ltri, rtri)
    return gt | (eq & (eq_rank < need_f))


def _compact(slot_masked, rows, chunk_rows, n_slots, vrows):
    p = len(vrows)
    assert rows % chunk_rows == 0
    n_ch = rows // chunk_rows
    cw = chunk_rows * 128
    iota_s = jax.lax.broadcasted_iota(jnp.int32, (n_slots, 1), 0).astype(jnp.int16)
    slot16 = slot_masked.astype(jnp.int16)
    one_b = jnp.bfloat16(1.0)
    zero_b = jnp.bfloat16(0.0)

    colacc = jnp.zeros((n_slots, 3 * p), jnp.float32)
    for ch in range(n_ch):
        r0 = ch * chunk_rows
        sl = slot16[r0:r0 + chunk_rows, :].reshape(1, cw)
        oh = jnp.where(sl == iota_s, one_b, zero_b)
        vs = [v[r0:r0 + chunk_rows, :].reshape(1, cw) for v in vrows]
        vm = jnp.concatenate(vs, axis=0)
        v1 = vm.astype(jnp.bfloat16)
        r1 = vm - v1.astype(jnp.float32)
        v2 = r1.astype(jnp.bfloat16)
        v3 = (r1 - v2.astype(jnp.float32)).astype(jnp.bfloat16)
        vcat = jnp.concatenate([v1, v2, v3], axis=0)
        colacc = colacc + jax.lax.dot_general(
            oh, vcat, (((1,), (1,)), ((), ())),
            preferred_element_type=jnp.float32)
    col = colacc[:, 0:p] + colacc[:, p:2 * p] + colacc[:, 2 * p:3 * p]
    return col, jnp.transpose(col)


def _nms_kernel(lg_ref, bx_ref, os_ref, ob_ref):
    ltri = _lane_tri()
    rtri = _row_tri(_R)
    lg = lg_ref[0, 0, :].reshape(_R, 128)
    sig = jax.nn.sigmoid(lg)
    key = jax.lax.bitcast_convert_type(sig, jnp.int32)
    thr = _radix_kth_largest(key, _K, signed=False)
    sel = _select_mask(key, thr, _K, ltri, rtri)
    slot = _excl_prefix(sel.astype(jnp.float32), ltri, rtri)
    slot_m = jnp.where(sel, slot, -1.0)

    fe = (jax.lax.broadcasted_iota(jnp.int32, (_R, 128), 0) * 128
          + jax.lax.broadcasted_iota(jnp.int32, (_R, 128), 1)).astype(jnp.float32)
    bx = bx_ref[0]
    planes = [sig,
              bx[0, :].reshape(_R, 128), bx[1, :].reshape(_R, 128),
              bx[2, :].reshape(_R, 128), bx[3, :].reshape(_R, 128),
              fe]
    col, row = _compact(slot_m, _R, 16, _S, planes)
    sig_c, sig_r = col[:, 0:1], row[0:1, :]
    x1c, y1c, x2c, y2c = col[:, 1:2], col[:, 2:3], col[:, 3:4], col[:, 4:5]
    x1r, y1r, x2r, y2r = row[1:2, :], row[2:3, :], row[3:4, :], row[4:5, :]
    fec, fer = col[:, 5:6], row[5:6, :]

    area_c = jnp.maximum(x2c - x1c, 0.0) * jnp.maximum(y2c - y1c, 0.0)
    area_r = jnp.maximum(x2r - x1r, 0.0) * jnp.maximum(y2r - y1r, 0.0)
    ix1 = jnp.maximum(x1c, x1r)
    iy1 = jnp.maximum(y1c, y1r)
    ix2 = jnp.minimum(x2c, x2r)
    iy2 = jnp.minimum(y2c, y2r)
    inter = jnp.maximum(ix2 - ix1, 0.0) * jnp.maximum(iy2 - iy1, 0.0)
    union = area_c + area_r - inter
    iou = jnp.where(union > 0.0, inter / union, 0.0)

    prec = (sig_c > sig_r) | ((sig_c == sig_r) & (fec < fer))
    m = (prec & (iou > _IOU_THRESH)).astype(jnp.float32)
    valid = (sig_r > _CONF_THRESH).astype(jnp.float32)

    def cond(state):
        return state[1]

    def body(state):
        keep, _ = state
        s1 = jnp.dot(keep, m, preferred_element_type=jnp.float32)
        k1 = valid * (s1 == 0.0).astype(jnp.float32)
        s2 = jnp.dot(k1, m, preferred_element_type=jnp.float32)
        k2 = valid * (s2 == 0.0).astype(jnp.float32)
        changed = jnp.any(k2 != k1)
        return (k2, changed)

    keep, _ = jax.lax.while_loop(cond, body, (valid, True))
    os_ref[0] = jnp.where(keep > 0.0, sig_r, _NEG)
    ob_ref[0] = row[1:5, :]


def _merge_kernel(sc_ref, bx_ref, ob_ref, os_ref, oc_ref, nv_ref):
    ltri = _lane_tri()
    rtri = _row_tri(_R3)
    s2 = sc_ref[0, 0, :].reshape(_R3, 128)
    u = jax.lax.bitcast_convert_type(s2, jnp.int32)
    key = u ^ ((u >> 31) & jnp.int32(0x7FFFFFFF))
    thr = _radix_kth_largest(key, _K3, signed=True)
    sel = _select_mask(key, thr, _K3, ltri, rtri)
    slot = _excl_prefix(sel.astype(jnp.float32), ltri, rtri)
    slot_m = jnp.where(sel, slot, -1.0)

    fe = (jax.lax.broadcasted_iota(jnp.int32, (_R3, 128), 0) * 128
          + jax.lax.broadcasted_iota(jnp.int32, (_R3, 128), 1))
    cls = (fe // _S).astype(jnp.float32)
    bx = bx_ref[0]
    planes = [s2,
              bx[0, :].reshape(_R3, 128), bx[1, :].reshape(_R3, 128),
              bx[2, :].reshape(_R3, 128), bx[3, :].reshape(_R3, 128),
              cls]
    col, row = _compact(slot_m, _R3, 8, _K3, planes)
    sig_c, sig_r = col[:, 0:1], row[0:1, :]
    mi = jax.lax.broadcasted_iota(jnp.int32, (_K3, 1), 0)
    mj = jax.lax.broadcasted_iota(jnp.int32, (1, _K3), 1)
    prec = (sig_c > sig_r) | ((sig_c == sig_r) & (mi < mj))
    rank = jnp.sum(prec.astype(jnp.float32), axis=0, keepdims=True)
    oh2 = (rank == jax.lax.broadcasted_iota(jnp.int32, (_K3, 1), 0)
           .astype(jnp.float32)).astype(jnp.bfloat16)
    c1 = col.astype(jnp.bfloat16)
    cr1 = col - c1.astype(jnp.float32)
    c2 = cr1.astype(jnp.bfloat16)
    c3 = (cr1 - c2.astype(jnp.float32)).astype(jnp.bfloat16)
    ccat = jnp.concatenate([c1, c2, c3], axis=1)
    sp = jnp.dot(oh2, ccat, preferred_element_type=jnp.float32)
    np_ = col.shape[1]
    sorted_pl = sp[:, 0:np_] + sp[:, np_:2 * np_] + sp[:, 2 * np_:3 * np_]
    score_s = sorted_pl[:, 0:1]
    vmask = score_s != _NEG
    ob_ref[0] = jnp.where(vmask, sorted_pl[:, 1:5], 0.0)
    os_ref[0] = jnp.where(vmask, score_s, 0.0)
    oc_ref[0] = jnp.where(vmask, sorted_pl[:, 5:6], 0.0)
    nfin = jnp.sum((s2 != _NEG).astype(jnp.float32))
    nv_ref[0] = jnp.zeros((1, 128), jnp.float32) + jnp.minimum(nfin, float(_MAX_TOTAL))


def kernel(images, predictions):
    B = predictions.shape[0]
    H, W = images.shape[1], images.shape[2]
    A = predictions.shape[1]
    anc = _anchors_np(H, W)
    anc_t = np.zeros((4, _AP), np.float32)
    anc_t[:, :A] = anc.T
    anc_t = jnp.asarray(anc_t)

    pred_t = jnp.transpose(predictions, (0, 2, 1))
    box_p = jnp.pad(pred_t[:, :4, :], ((0, 0), (0, 0), (0, _AP - A)))
    logits = jnp.pad(pred_t[:, 4:, :], ((0, 0), (0, 0), (0, _AP - A)),
                     constant_values=-1e30)
    logits = logits.reshape(B * _NUM_CLASSES, 1, _AP)

    corners = pl.pallas_call(
        _decode_kernel,
        out_shape=jax.ShapeDtypeStruct((B, 4, _AP), jnp.float32),
        grid=(B,),
        in_specs=[pl.BlockSpec((1, 4, _AP), lambda b: (b, 0, 0)),
                  pl.BlockSpec((4, _AP), lambda b: (0, 0))],
        out_specs=pl.BlockSpec((1, 4, _AP), lambda b: (b, 0, 0)),
    )(box_p, anc_t)

    nC = _NUM_CLASSES
    cls_scores, cls_boxes = pl.pallas_call(
        _nms_kernel,
        out_shape=(jax.ShapeDtypeStruct((B * nC, 1, _S), jnp.float32),
                   jax.ShapeDtypeStruct((B * nC, 4, _S), jnp.float32)),
        grid=(B * nC,),
        in_specs=[pl.BlockSpec((1, 1, _AP), lambda i: (i, 0, 0)),
                  pl.BlockSpec((1, 4, _AP), lambda i: (i // nC, 0, 0))],
        out_specs=(pl.BlockSpec((1, 1, _S), lambda i: (i, 0, 0)),
                   pl.BlockSpec((1, 4, _S), lambda i: (i, 0, 0))),
    )(logits, corners)

    scores_m = cls_scores.reshape(B, 1, _M3)
    boxes_m = (cls_boxes.reshape(B, nC, 4, _S)
               .transpose(0, 2, 1, 3).reshape(B, 4, _M3))

    ob, osc, ocl, nv = pl.pallas_call(
        _merge_kernel,
        out_shape=(jax.ShapeDtypeStruct((B, _K3, 4), jnp.float32),
                   jax.ShapeDtypeStruct((B, _K3, 1), jnp.float32),
                   jax.ShapeDtypeStruct((B, _K3, 1), jnp.float32),
                   jax.ShapeDtypeStruct((B, 1, 128), jnp.float32)),
        grid=(B,),
        in_specs=[pl.BlockSpec((1, 1, _M3), lambda b: (b, 0, 0)),
                  pl.BlockSpec((1, 4, _M3), lambda b: (b, 0, 0))],
        out_specs=(pl.BlockSpec((1, _K3, 4), lambda b: (b, 0, 0)),
                   pl.BlockSpec((1, _K3, 1), lambda b: (b, 0, 0)),
                   pl.BlockSpec((1, _K3, 1), lambda b: (b, 0, 0)),
                   pl.BlockSpec((1, 1, 128), lambda b: (b, 0, 0))),
    )(scores_m, boxes_m)

    out_boxes = ob[:, :_MAX_TOTAL, :]
    out_scores = osc[:, :_MAX_TOTAL, 0]
    out_classes = ocl[:, :_MAX_TOTAL, 0]
    num_valid = nv[:, 0, 0].astype(jnp.int32)
    return out_boxes, out_scores, out_classes, num_valid


# speedup vs baseline: 10.3807x; 1.0001x over previous
import math

import jax
import jax.numpy as jnp
import numpy as np
from jax.experimental import pallas as pl

_NUM_CLASSES = 18
_CONF_THRESH = 0.05
_IOU_THRESH = 0.5
_K = 500
_S = 512
_K3 = 1024
_MAX_TOTAL = 1000
_A = 49104
_AP = 49152
_R = 384
_M3 = _NUM_CLASSES * _S
_R3 = 72
_NEG = -1.0


def _anchors_np(image_height, image_width):
    aspect_ratios = [0.5, 1.0, 2.0]
    scales = [2 ** x for x in [0.0, 1.0 / 3.0, 2.0 / 3.0]]
    areas = [x ** 2 for x in [32.0, 64.0, 128.0, 256.0, 512.0]]
    strides = [2 ** i for i in range(3, 8)]
    anchors = []
    for li, stride in enumerate(strides):
        area = areas[li]
        dims = []
        for ratio in aspect_ratios:
            h = math.sqrt(area / ratio)
            w = area / h
            for s in scales:
                dims.append([s * w, s * h])
        dims = np.array(dims, dtype=np.float32)
        fh = int(math.ceil(image_height / stride))
        fw = int(math.ceil(image_width / stride))
        rx = (np.arange(fw, dtype=np.float32) + 0.5) * stride
        ry = (np.arange(fh, dtype=np.float32) + 0.5) * stride
        cx, cy = np.meshgrid(rx, ry)
        centers = np.stack([cx, cy], axis=-1)
        centers = np.tile(centers[:, :, None, :], [1, 1, 9, 1])
        d = np.tile(dims[None, None, :, :], [fh, fw, 1, 1])
        a = np.concatenate([centers, d], axis=-1).reshape(-1, 4)
        anchors.append(a)
    return np.concatenate(anchors, axis=0)


def _decode_kernel(bp_ref, an_ref, out_ref):
    t = bp_ref[0]
    a = an_ref[...]
    tx = t[0:1, :] * 0.1
    ty = t[1:2, :] * 0.1
    tw = t[2:3, :] * 0.2
    th = t[3:4, :] * 0.2
    cx = tx * a[2:3, :] + a[0:1, :]
    cy = ty * a[3:4, :] + a[1:2, :]
    w = jnp.exp(tw) * a[2:3, :]
    h = jnp.exp(th) * a[3:4, :]
    out_ref[0, 0:1, :] = cx - w / 2.0
    out_ref[0, 1:2, :] = cy - h / 2.0
    out_ref[0, 2:3, :] = cx + w / 2.0
    out_ref[0, 3:4, :] = cy + h / 2.0


def _lane_tri():
    r = jax.lax.broadcasted_iota(jnp.int32, (128, 128), 0)
    c = jax.lax.broadcasted_iota(jnp.int32, (128, 128), 1)
    return (r < c).astype(jnp.float32)


def _row_tri(n):
    r = jax.lax.broadcasted_iota(jnp.int32, (n, n), 0)
    c = jax.lax.broadcasted_iota(jnp.int32, (n, n), 1)
    return (c < r).astype(jnp.float32)


def _excl_prefix(mask_f, ltri, rtri):
    lane_p = jnp.dot(mask_f, ltri, preferred_element_type=jnp.float32)
    row_tot = jnp.sum(mask_f, axis=1, keepdims=True)
    row_base = jnp.dot(rtri, row_tot, preferred_element_type=jnp.float32)
    return lane_p + row_base


def _radix_kth_largest(key, k, signed):
    if signed:
        cnt0 = jnp.sum((key >= 0).astype(jnp.int32))
        base0 = jnp.where(cnt0 >= k, jnp.int32(0), jnp.int32(-2147483648))
    else:
        base0 = jnp.int32(0)

    def body(b, base):
        cand = base | (jnp.int32(1) << (30 - b))
        cnt = jnp.sum((key >= cand).astype(jnp.int32))
        return jnp.where(cnt >= k, cand, base)

    return jax.lax.fori_loop(0, 31, body, base0)


def _select_mask(key, thr, k, ltri, rtri):
    gt = key > thr
    eq = key == thr
    g = jnp.sum(gt.astype(jnp.int32))
    need_f = (k - g).astype(jnp.float32)
    eq_rank = _excl_prefix(eq.astype(jnp.float32), ltri, rtri)
    return gt | (eq & (eq_rank < need_f))


def _compact(slot_masked, rows, chunk_rows, n_slots, vrows):
    p = len(vrows)
    assert rows % chunk_rows == 0
    n_ch = rows // chunk_rows
    cw = chunk_rows * 128
    iota_s = jax.lax.broadcasted_iota(jnp.int32, (n_slots, 1), 0).astype(jnp.int16)
    slot16 = slot_masked.astype(jnp.int16)
    one_b = jnp.bfloat16(1.0)
    zero_b = jnp.bfloat16(0.0)

    colacc = jnp.zeros((n_slots, 3 * p), jnp.float32)
    for ch in range(n_ch):
        r0 = ch * chunk_rows
        sl = slot16[r0:r0 + chunk_rows, :].reshape(1, cw)
        oh = jnp.where(sl == iota_s, one_b, zero_b)
        vs = [v[r0:r0 + chunk_rows, :].reshape(1, cw) for v in vrows]
        vm = jnp.concatenate(vs, axis=0)
        v1 = vm.astype(jnp.bfloat16)
        r1 = vm - v1.astype(jnp.float32)
        v2 = r1.astype(jnp.bfloat16)
        v3 = (r1 - v2.astype(jnp.float32)).astype(jnp.bfloat16)
        vcat = jnp.concatenate([v1, v2, v3], axis=0)
        colacc = colacc + jax.lax.dot_general(
            oh, vcat, (((1,), (1,)), ((), ())),
            preferred_element_type=jnp.float32)
    col = colacc[:, 0:p] + colacc[:, p:2 * p] + colacc[:, 2 * p:3 * p]
    return col, jnp.transpose(col)


def _nms_kernel(lg_ref, bx_ref, os_ref, ob_ref):
    ltri = _lane_tri()
    rtri = _row_tri(_R)
    lg = lg_ref[0, 0, :].reshape(_R, 128)
    sig = jax.nn.sigmoid(lg)
    key = jax.lax.bitcast_convert_type(sig, jnp.int32)
    thr = _radix_kth_largest(key, _K, signed=False)
    sel = _select_mask(key, thr, _K, ltri, rtri)
    slot = _excl_prefix(sel.astype(jnp.float32), ltri, rtri)
    slot_m = jnp.where(sel, slot, -1.0)

    fe = (jax.lax.broadcasted_iota(jnp.int32, (_R, 128), 0) * 128
          + jax.lax.broadcasted_iota(jnp.int32, (_R, 128), 1)).astype(jnp.float32)
    bx = bx_ref[0]
    planes = [sig,
              bx[0, :].reshape(_R, 128), bx[1, :].reshape(_R, 128),
              bx[2, :].reshape(_R, 128), bx[3, :].reshape(_R, 128),
              fe]
    col, row = _compact(slot_m, _R, 16, _S, planes)
    sig_c, sig_r = col[:, 0:1], row[0:1, :]
    x1c, y1c, x2c, y2c = col[:, 1:2], col[:, 2:3], col[:, 3:4], col[:, 4:5]
    x1r, y1r, x2r, y2r = row[1:2, :], row[2:3, :], row[3:4, :], row[4:5, :]
    fec, fer = col[:, 5:6], row[5:6, :]

    area_c = jnp.maximum(x2c - x1c, 0.0) * jnp.maximum(y2c - y1c, 0.0)
    area_r = jnp.maximum(x2r - x1r, 0.0) * jnp.maximum(y2r - y1r, 0.0)
    ix1 = jnp.maximum(x1c, x1r)
    iy1 = jnp.maximum(y1c, y1r)
    ix2 = jnp.minimum(x2c, x2r)
    iy2 = jnp.minimum(y2c, y2r)
    inter = jnp.maximum(ix2 - ix1, 0.0) * jnp.maximum(iy2 - iy1, 0.0)
    union = area_c + area_r - inter
    iou = jnp.where(union > 0.0, inter / union, 0.0)

    prec = (sig_c > sig_r) | ((sig_c == sig_r) & (fec < fer))
    m = (prec & (iou > _IOU_THRESH)).astype(jnp.float32)
    valid = (sig_r > _CONF_THRESH).astype(jnp.float32)

    def cond(state):
        return state[1]

    def body(state):
        keep, _ = state
        s1 = jnp.dot(keep, m, preferred_element_type=jnp.float32)
        k1 = valid * (s1 == 0.0).astype(jnp.float32)
        s2 = jnp.dot(k1, m, preferred_element_type=jnp.float32)
        k2 = valid * (s2 == 0.0).astype(jnp.float32)
        changed = jnp.any(k2 != k1)
        return (k2, changed)

    keep, _ = jax.lax.while_loop(cond, body, (valid, True))
    os_ref[0] = jnp.where(keep > 0.0, sig_r, _NEG)
    ob_ref[0] = row[1:5, :]


def _merge_kernel(sc_ref, bx_ref, ob_ref, os_ref, oc_ref, nv_ref):
    ltri = _lane_tri()
    rtri = _row_tri(_R3)
    s2 = sc_ref[0, 0, :].reshape(_R3, 128)
    u = jax.lax.bitcast_convert_type(s2, jnp.int32)
    key = u ^ ((u >> 31) & jnp.int32(0x7FFFFFFF))
    thr = _radix_kth_largest(key, _K3, signed=True)
    sel = _select_mask(key, thr, _K3, ltri, rtri)
    slot = _excl_prefix(sel.astype(jnp.float32), ltri, rtri)
    slot_m = jnp.where(sel, slot, -1.0)

    fe = (jax.lax.broadcasted_iota(jnp.int32, (_R3, 128), 0) * 128
          + jax.lax.broadcasted_iota(jnp.int32, (_R3, 128), 1))
    cls = (fe // _S).astype(jnp.float32)
    bx = bx_ref[0]
    planes = [s2,
              bx[0, :].reshape(_R3, 128), bx[1, :].reshape(_R3, 128),
              bx[2, :].reshape(_R3, 128), bx[3, :].reshape(_R3, 128),
              cls]
    col, row = _compact(slot_m, _R3, 8, _K3, planes)
    sig_c, sig_r = col[:, 0:1], row[0:1, :]
    mi = jax.lax.broadcasted_iota(jnp.int32, (_K3, 1), 0)
    mj = jax.lax.broadcasted_iota(jnp.int32, (1, _K3), 1)
    prec = (sig_c > sig_r) | ((sig_c == sig_r) & (mi < mj))
    rank = jnp.sum(prec.astype(jnp.float32), axis=0, keepdims=True)
    oh2 = (rank == jax.lax.broadcasted_iota(jnp.int32, (_K3, 1), 0)
           .astype(jnp.float32)).astype(jnp.bfloat16)
    c1 = col.astype(jnp.bfloat16)
    cr1 = col - c1.astype(jnp.float32)
    c2 = cr1.astype(jnp.bfloat16)
    c3 = (cr1 - c2.astype(jnp.float32)).astype(jnp.bfloat16)
    ccat = jnp.concatenate([c1, c2, c3], axis=1)
    sp = jnp.dot(oh2, ccat, preferred_element_type=jnp.float32)
    np_ = col.shape[1]
    sorted_pl = sp[:, 0:np_] + sp[:, np_:2 * np_] + sp[:, 2 * np_:3 * np_]
    score_s = sorted_pl[:, 0:1]
    vmask = score_s != _NEG
    ob_ref[0] = jnp.where(vmask, sorted_pl[:, 1:5], 0.0)
    os_ref[0] = jnp.where(vmask, score_s, 0.0)
    oc_ref[0] = jnp.where(vmask, sorted_pl[:, 5:6], 0.0)
    nfin = jnp.sum((s2 != _NEG).astype(jnp.float32))
    nv_ref[0] = jnp.zeros((1, 128), jnp.float32) + jnp.minimum(nfin, float(_MAX_TOTAL))


def kernel(images, predictions):
    B = predictions.shape[0]
    H, W = images.shape[1], images.shape[2]
    A = predictions.shape[1]
    anc = _anchors_np(H, W)
    anc_t = np.zeros((4, _AP), np.float32)
    anc_t[:, :A] = anc.T
    anc_t = jnp.asarray(anc_t)

    pred_t = jnp.transpose(predictions, (0, 2, 1))
    box_p = jnp.pad(pred_t[:, :4, :], ((0, 0), (0, 0), (0, _AP - A)))
    logits = jnp.pad(pred_t[:, 4:, :], ((0, 0), (0, 0), (0, _AP - A)),
                     constant_values=-1e30)
    logits = logits.reshape(B * _NUM_CLASSES, 1, _AP)

    corners = pl.pallas_call(
        _decode_kernel,
        out_shape=jax.ShapeDtypeStruct((B, 4, _AP), jnp.float32),
        grid=(B,),
        in_specs=[pl.BlockSpec((1, 4, _AP), lambda b: (b, 0, 0)),
                  pl.BlockSpec((4, _AP), lambda b: (0, 0))],
        out_specs=pl.BlockSpec((1, 4, _AP), lambda b: (b, 0, 0)),
    )(box_p, anc_t)

    nC = _NUM_CLASSES
    cls_scores, cls_boxes = pl.pallas_call(
        _nms_kernel,
        out_shape=(jax.ShapeDtypeStruct((B * nC, 1, _S), jnp.float32),
                   jax.ShapeDtypeStruct((B * nC, 4, _S), jnp.float32)),
        grid=(B * nC,),
        in_specs=[pl.BlockSpec((1, 1, _AP), lambda i: (i, 0, 0)),
                  pl.BlockSpec((1, 4, _AP), lambda i: (i // nC, 0, 0))],
        out_specs=(pl.BlockSpec((1, 1, _S), lambda i: (i, 0, 0)),
                   pl.BlockSpec((1, 4, _S), lambda i: (i, 0, 0))),
    )(logits, corners)

    scores_m = cls_scores.reshape(B, 1, _M3)
    boxes_m = (cls_boxes.reshape(B, nC, 4, _S)
               .transpose(0, 2, 1, 3).reshape(B, 4, _M3))

    ob, osc, ocl, nv = pl.pallas_call(
        _merge_kernel,
        out_shape=(jax.ShapeDtypeStruct((B, _K3, 4), jnp.float32),
                   jax.ShapeDtypeStruct((B, _K3, 1), jnp.float32),
                   jax.ShapeDtypeStruct((B, _K3, 1), jnp.float32),
                   jax.ShapeDtypeStruct((B, 1, 128), jnp.float32)),
        grid=(B,),
        in_specs=[pl.BlockSpec((1, 1, _M3), lambda b: (b, 0, 0)),
                  pl.BlockSpec((1, 4, _M3), lambda b: (b, 0, 0))],
        out_specs=(pl.BlockSpec((1, _K3, 4), lambda b: (b, 0, 0)),
                   pl.BlockSpec((1, _K3, 1), lambda b: (b, 0, 0)),
                   pl.BlockSpec((1, _K3, 1), lambda b: (b, 0, 0)),
                   pl.BlockSpec((1, 1, 128), lambda b: (b, 0, 0))),
    )(scores_m, boxes_m)

    out_boxes = ob[:, :_MAX_TOTAL, :]
    out_scores = osc[:, :_MAX_TOTAL, 0]
    out_classes = ocl[:, :_MAX_TOTAL, 0]
    num_valid = nv[:, 0, 0].astype(jnp.int32)
    return out_boxes, out_scores, out_classes, num_valid
